# baseline (speedup 1.0000x reference)
import functools

import jax
import jax.numpy as jnp
from jax import lax
from jax.experimental import pallas as pl
from jax.experimental.pallas import tpu as pltpu
from jax.experimental.pallas import tpu_sc as plsc


def _sc_gather(table128, src_pad, dst_pad, *, B, W, N, RE, Ep):
    CH = Ep // 128
    CELLS = 128 // RE
    mesh = plsc.VectorSubcoreMesh(core_axis_name="c", subcore_axis_name="s")
    NC = mesh.num_cores

    @functools.partial(
        pl.kernel,
        out_type=jax.ShapeDtypeStruct((B, CH, 128, 128), jnp.float32),
        mesh=mesh,
        scratch_types=[
            pltpu.VMEM((Ep,), jnp.int32),
            pltpu.VMEM((Ep,), jnp.int32),
            pltpu.VMEM((CH, 128), jnp.int32),
            pltpu.VMEM((CH, 128, 128), jnp.float32),
            pltpu.SemaphoreType.DMA,
        ],
    )
    def k(table_h, src_h, dst_h, out_h, src_v, dst_v, idx_v, rows_v, sem):
        wid = lax.axis_index("s") * NC + lax.axis_index("c")

        @pl.when(wid < B)
        def _():
            pltpu.sync_copy(src_h, src_v)
            pltpu.sync_copy(dst_h, dst_v)
            base = (wid * W + (W - 1)) * (N * CELLS)
            for j in range(CH):
                for t in range(128 // 16):
                    off = j * 128 + t * 16
                    sv = src_v[pl.ds(off, 16)]
                    dv = dst_v[pl.ds(off, 16)]
                    idx_v[j, pl.ds(t * 16, 16)] = (
                        sv * CELLS
                        + lax.shift_right_logical(dv, CELLS.bit_length() - 1)
                        + base)
            copies = [
                pltpu.make_async_copy(table_h.at[idx_v.at[j]], rows_v.at[j], sem)
                for j in range(CH)
            ]
            for c in copies:
                c.start()
            for c in copies:
                c.wait()
            pltpu.sync_copy(rows_v, out_h.at[wid])

    return k(table128, src_pad, dst_pad)


def _extract_body(ge128_r, dstmod_r, out_r, *, RE):
    ge128 = ge128_r[...]
    dstmod = dstmod_r[...]
    acc = jnp.zeros((ge128.shape[0], RE), jnp.float32)
    for g in range(128 // RE):
        sel = (dstmod == g).astype(jnp.float32)
        acc = acc + ge128[:, g * RE:(g + 1) * RE] * sel
    out_r[...] = acc


def _extract(ge128, dstmod, *, B, Ep, RE):
    return pl.pallas_call(
        functools.partial(_extract_body, RE=RE),
        grid=(B,),
        in_specs=[
            pl.BlockSpec((Ep, 128), lambda b: (b, 0)),
            pl.BlockSpec((Ep, 1), lambda b: (0, 0)),
        ],
        out_specs=pl.BlockSpec((Ep, RE), lambda b: (b, 0)),
        out_shape=jax.ShapeDtypeStruct((B * Ep, RE), jnp.float32),
    )(ge128, dstmod)


def _tc_body(refs, B, N, Ep):
    (dn_r, dl_r, ge_r, graph_r, srcc_r, dstc_r, t2p_r,
     Wne, bne, Wle, ble, Wgn, Wgl, Wee, bee, We2n,
     Wn, Wl, We, Wdn, Wzn, Wde, Wze, Wdl, Wzl,
     Wdno, bdno, Wdeo, bdeo, Wdlo, bdlo,
     W1n, W1l, W1e, bs1, wdiff, bdiff,
     out0_r, out1_r) = refs

    def dot(a, b):
        return jnp.dot(a, b, preferred_element_type=jnp.float32)

    def dot_c00(a, b):
        return lax.dot_general(a, b, (((0,), (0,)), ((), ())),
                               preferred_element_type=jnp.float32)

    dn = dn_r[...]
    dl = dl_r[...]
    ge = ge_r[...]

    xn = dot(dn, Wne[...]) + bne[...]
    xl = dot(dl, Wle[...]) + ble[...]
    xe = dot(ge, Wee[...]) + bee[...]

    s_dn = dn[0:N]
    s_dl = dl[0:N]
    for b in range(1, B):
        s_dn = s_dn + dn[b * N:(b + 1) * N]
        s_dl = s_dl + dl[b * N:(b + 1) * N]
    s_node = dot(s_dn / B, Wne[...]) + bne[...]
    s_log = dot(s_dl / B, Wle[...]) + ble[...]
    h = jnp.tanh(dot(s_node, Wgn[...]) + dot(s_log, Wgl[...]))
    hhT = lax.dot_general(h, h, (((1,), (1,)), ((), ())),
                          preferred_element_type=jnp.float32)
    ewmat = (1.0 / (1.0 + jnp.exp(-hhT))) * graph_r[...]

    iota_e = lax.broadcasted_iota(jnp.int32, (Ep, N), 1)
    oh_src = (srcc_r[...] == iota_e).astype(jnp.float32)
    oh_dst = (dstc_r[...] == iota_e).astype(jnp.float32)
    ew_col = jnp.sum(dot(oh_src, ewmat) * oh_dst, axis=1, keepdims=True)

    t2p = t2p_r[...]

    ew_full = jnp.concatenate([ew_col] * B, axis=0)
    e_fl = xe * ew_full
    msg = dot(e_fl, We2n[...])
    z_edge = jnp.tanh(dot(e_fl, We[...]))
    edge = jnp.tanh(dot(xe, Wde[...]) + dot(z_edge, Wze[...]))
    rec1 = jnp.square(dot(edge, Wdeo[...]) + bdeo[...] - ge)

    aggs = []
    rec_edges = []
    for b in range(B):
        aggs.append(dot_c00(oh_dst, msg[b * Ep:(b + 1) * Ep]))
        rec_edges.append(dot_c00(t2p, rec1[b * Ep:(b + 1) * Ep]))
    agg = jnp.concatenate(aggs, axis=0)
    rec_edge = jnp.concatenate(rec_edges, axis=0)

    z_node = jnp.tanh(dot(xn, Wn[...]) + agg)
    z_log = jnp.tanh(dot(xl, Wl[...]))
    node = jnp.tanh(dot(xn, Wdn[...]) + dot(z_node, Wzn[...]))
    logf = jnp.tanh(dot(xl, Wdl[...]) + dot(z_log, Wzl[...]))
    rec_node = jnp.square(dot(node, Wdno[...]) + bdno[...] - dn)
    rec_log = jnp.square(dot(logf, Wdlo[...]) + bdlo[...] - dl)

    pre = (dot(rec_node, W1n[...]) + dot(rec_log, W1l[...])
           + dot(rec_edge, W1e[...]) + bs1[...])
    hshow = jnp.where(pre > 0, pre, 0.01 * pre)
    delta = dot(hshow, wdiff[...]) + bdiff[...]
    out0_r[...] = 1.0 / (1.0 + jnp.exp(-delta))
    out1_r[...] = 1.0 / (1.0 + jnp.exp(delta))


def kernel(data_node, data_edge, data_log, groundtruth_cls, graph, src, dst,
           trace2pod, params):
    p = params
    B, W, N, RN = data_node.shape
    RE = data_edge.shape[-1]
    LL = data_log.shape[-1]
    E = src.shape[0]
    Ep = ((E + 127) // 128) * 128

    src_pad = jnp.zeros((Ep,), jnp.int32).at[:E].set(src.astype(jnp.int32))
    dst_pad = jnp.zeros((Ep,), jnp.int32).at[:E].set(dst.astype(jnp.int32))

    table128 = data_edge.reshape(-1, 128)
    ge128 = _sc_gather(table128, src_pad, dst_pad, B=B, W=W, N=N, RE=RE, Ep=Ep)
    dstmod = jnp.full((Ep, 1), 128 // RE, jnp.int32).at[:E, 0].set(
        dst.astype(jnp.int32) % (128 // RE))
    ge2 = _extract(ge128.reshape(B * Ep, 128), dstmod, B=B, Ep=Ep, RE=RE)

    src_col = jnp.full((Ep, 1), N, jnp.int32).at[:E, 0].set(src.astype(jnp.int32))
    dst_col = jnp.full((Ep, 1), N, jnp.int32).at[:E, 0].set(dst.astype(jnp.int32))
    t2p_pad = jnp.zeros((Ep, N), jnp.float32).at[:E].set(trace2pod)

    dn2 = data_node[:, -1].reshape(B * N, RN)
    dl2 = data_log[:, -1].reshape(B * N, LL)

    r2 = lambda v: v.reshape(1, -1)
    ins = [
        dn2, dl2, ge2, graph, src_col, dst_col, t2p_pad,
        p['Wne'], r2(p['bne']), p['Wle'], r2(p['ble']),
        p['Wgn'], p['Wgl'], p['Wee'], r2(p['bee']), p['We2n'],
        p['Wn'], p['Wl'], p['We'], p['Wdn'], p['Wzn'],
        p['Wde'], p['Wze'], p['Wdl'], p['Wzl'],
        p['Wdn_out'], r2(p['bdn_out']), p['Wde_out'], r2(p['bde_out']),
        p['Wdl_out'], r2(p['bdl_out']),
        p['Ws1'][:RN], p['Ws1'][RN:RN + LL], p['Ws1'][RN + LL:], r2(p['bs1']),
        (p['Ws2'][:, 0] - p['Ws2'][:, 1]).reshape(-1, 1),
        (p['bs2'][0] - p['bs2'][1]).reshape(1, 1),
    ]

    body = lambda *refs: _tc_body(refs, B, N, Ep)
    out0, out1 = pl.pallas_call(
        body,
        out_shape=[
            jax.ShapeDtypeStruct((B * N, 1), jnp.float32),
            jax.ShapeDtypeStruct((B * N, 1), jnp.float32),
        ],
    )(*ins)

    return jnp.concatenate([out0, out1], axis=1).reshape(B, N, 2)


# speedup vs baseline: 1.0739x; 1.0739x over previous
import functools

import jax
import jax.numpy as jnp
from jax import lax
from jax.experimental import pallas as pl
from jax.experimental.pallas import tpu as pltpu
from jax.experimental.pallas import tpu_sc as plsc


def _sc_gather(table_flat, src_pad, dst_pad, *, B, N, RE, Ep):
    CH = Ep // 128
    mesh = plsc.VectorSubcoreMesh(core_axis_name="c", subcore_axis_name="s")
    NC = mesh.num_cores

    @functools.partial(
        pl.kernel,
        out_type=jax.ShapeDtypeStruct((B, CH, 128, RE), jnp.float32),
        mesh=mesh,
        scratch_types=[
            pltpu.VMEM((Ep,), jnp.int32),
            pltpu.VMEM((Ep,), jnp.int32),
            pltpu.VMEM((CH, 128), jnp.int32),
            pltpu.VMEM((CH, 128, RE), jnp.float32),
            pltpu.SemaphoreType.DMA,
        ],
        compiler_params=pltpu.CompilerParams(use_tc_tiling_on_sc=False),
    )
    def k(table_h, src_h, dst_h, out_h, src_v, dst_v, idx_v, rows_v, sem):
        wid = lax.axis_index("s") * NC + lax.axis_index("c")

        @pl.when(wid < B)
        def _():
            pltpu.sync_copy(src_h, src_v)
            pltpu.sync_copy(dst_h, dst_v)
            base = wid * (N * N)
            for j in range(CH):
                for t in range(128 // 16):
                    off = j * 128 + t * 16
                    sv = src_v[pl.ds(off, 16)]
                    dv = dst_v[pl.ds(off, 16)]
                    idx_v[j, pl.ds(t * 16, 16)] = sv * N + dv + base
            copies = [
                pltpu.make_async_copy(table_h.at[idx_v.at[j]], rows_v.at[j], sem)
                for j in range(CH)
            ]
            for c in copies:
                c.start()
            for c in copies:
                c.wait()
            pltpu.sync_copy(rows_v, out_h.at[wid])

    return k(table_flat, src_pad, dst_pad)


def _slice_edges(data_edge, *, B, W, N, RE):
    def body(de_r, out_r):
        out_r[...] = de_r[...].reshape(N * N, RE)

    return pl.pallas_call(
        body,
        grid=(B,),
        in_specs=[pl.BlockSpec((1, 1, N, N, RE),
                               lambda b: (b, W - 1, 0, 0, 0))],
        out_specs=pl.BlockSpec((N * N, RE), lambda b: (b, 0)),
        out_shape=jax.ShapeDtypeStruct((B * N * N, RE), jnp.float32),
    )(data_edge)


def _tc_body(refs, B, N, Ep):
    (dn_r, dl_r, ge_r, graph_r, srcc_r, dstc_r, t2p_r,
     Wne, bne, Wle, ble, Wgn, Wgl, Wee, bee, We2n,
     Wn, Wl, We, Wdn, Wzn, Wde, Wze, Wdl, Wzl,
     Wdno, bdno, Wdeo, bdeo, Wdlo, bdlo,
     W1n, W1l, W1e, bs1, wdiff, bdiff,
     out0_r, out1_r) = refs

    def dot(a, b):
        return jnp.dot(a, b, preferred_element_type=jnp.float32)

    def dot_c00(a, b):
        return lax.dot_general(a, b, (((0,), (0,)), ((), ())),
                               preferred_element_type=jnp.float32)

    BN = B * N
    dn = dn_r[...].reshape(BN, dn_r.shape[-1])
    dl = dl_r[...].reshape(BN, dl_r.shape[-1])
    ge = ge_r[...]

    xn = dot(dn, Wne[...]) + bne[...]
    xl = dot(dl, Wle[...]) + ble[...]
    xe = dot(ge, Wee[...]) + bee[...]

    s_dn = dn[0:N]
    s_dl = dl[0:N]
    for b in range(1, B):
        s_dn = s_dn + dn[b * N:(b + 1) * N]
        s_dl = s_dl + dl[b * N:(b + 1) * N]
    s_node = dot(s_dn / B, Wne[...]) + bne[...]
    s_log = dot(s_dl / B, Wle[...]) + ble[...]
    h = jnp.tanh(dot(s_node, Wgn[...]) + dot(s_log, Wgl[...]))
    hhT = lax.dot_general(h, h, (((1,), (1,)), ((), ())),
                          preferred_element_type=jnp.float32)
    ewmat = (1.0 / (1.0 + jnp.exp(-hhT))) * graph_r[...]

    iota_e = lax.broadcasted_iota(jnp.int32, (Ep, N), 1)
    oh_src = (srcc_r[...] == iota_e).astype(jnp.float32)
    oh_dst = (dstc_r[...] == iota_e).astype(jnp.float32)
    ew_col = jnp.sum(dot(oh_src, ewmat) * oh_dst, axis=1, keepdims=True)

    t2p = t2p_r[...]

    ew_full = jnp.concatenate([ew_col] * B, axis=0)
    e_fl = xe * ew_full
    msg = dot(e_fl, We2n[...])
    z_edge = jnp.tanh(dot(e_fl, We[...]))
    edge = jnp.tanh(dot(xe, Wde[...]) + dot(z_edge, Wze[...]))
    rec1 = jnp.square(dot(edge, Wdeo[...]) + bdeo[...] - ge)

    aggs = []
    rec_edges = []
    for b in range(B):
        aggs.append(dot_c00(oh_dst, msg[b * Ep:(b + 1) * Ep]))
        rec_edges.append(dot_c00(t2p, rec1[b * Ep:(b + 1) * Ep]))
    agg = jnp.concatenate(aggs, axis=0)
    rec_edge = jnp.concatenate(rec_edges, axis=0)

    z_node = jnp.tanh(dot(xn, Wn[...]) + agg)
    z_log = jnp.tanh(dot(xl, Wl[...]))
    node = jnp.tanh(dot(xn, Wdn[...]) + dot(z_node, Wzn[...]))
    logf = jnp.tanh(dot(xl, Wdl[...]) + dot(z_log, Wzl[...]))
    rec_node = jnp.square(dot(node, Wdno[...]) + bdno[...] - dn)
    rec_log = jnp.square(dot(logf, Wdlo[...]) + bdlo[...] - dl)

    pre = (dot(rec_node, W1n[...]) + dot(rec_log, W1l[...])
           + dot(rec_edge, W1e[...]) + bs1[...])
    hshow = jnp.where(pre > 0, pre, 0.01 * pre)
    delta = dot(hshow, wdiff[...]) + bdiff[...]
    out0_r[...] = 1.0 / (1.0 + jnp.exp(-delta))
    out1_r[...] = 1.0 / (1.0 + jnp.exp(delta))


def kernel(data_node, data_edge, data_log, groundtruth_cls, graph, src, dst,
           trace2pod, params):
    p = params
    B, W, N, RN = data_node.shape
    RE = data_edge.shape[-1]
    LL = data_log.shape[-1]
    E = src.shape[0]
    Ep = ((E + 127) // 128) * 128

    src_pad = jnp.zeros((Ep,), jnp.int32).at[:E].set(src.astype(jnp.int32))
    dst_pad = jnp.zeros((Ep,), jnp.int32).at[:E].set(dst.astype(jnp.int32))

    table_flat = _slice_edges(data_edge, B=B, W=W, N=N, RE=RE)
    ge = _sc_gather(table_flat, src_pad, dst_pad, B=B, N=N, RE=RE, Ep=Ep)
    ge2 = ge.reshape(B * Ep, RE)

    src_col = jnp.full((Ep, 1), N, jnp.int32).at[:E, 0].set(src.astype(jnp.int32))
    dst_col = jnp.full((Ep, 1), N, jnp.int32).at[:E, 0].set(dst.astype(jnp.int32))
    t2p_pad = jnp.zeros((Ep, N), jnp.float32).at[:E].set(trace2pod)

    r2 = lambda v: v.reshape(1, -1)
    ins = [
        data_node, data_log, ge2, graph, src_col, dst_col, t2p_pad,
        p['Wne'], r2(p['bne']), p['Wle'], r2(p['ble']),
        p['Wgn'], p['Wgl'], p['Wee'], r2(p['bee']), p['We2n'],
        p['Wn'], p['Wl'], p['We'], p['Wdn'], p['Wzn'],
        p['Wde'], p['Wze'], p['Wdl'], p['Wzl'],
        p['Wdn_out'], r2(p['bdn_out']), p['Wde_out'], r2(p['bde_out']),
        p['Wdl_out'], r2(p['bdl_out']),
        p['Ws1'][:RN], p['Ws1'][RN:RN + LL], p['Ws1'][RN + LL:], r2(p['bs1']),
        (p['Ws2'][:, 0] - p['Ws2'][:, 1]).reshape(-1, 1),
        (p['bs2'][0] - p['bs2'][1]).reshape(1, 1),
    ]

    def zero_map(nd):
        return lambda i: (0,) * nd

    in_specs = [
        pl.BlockSpec((B, 1, N, RN), lambda i: (0, W - 1, 0, 0)),
        pl.BlockSpec((B, 1, N, LL), lambda i: (0, W - 1, 0, 0)),
    ] + [pl.BlockSpec(x.shape, zero_map(x.ndim)) for x in ins[2:]]

    body = lambda *refs: _tc_body(refs, B, N, Ep)
    out0, out1 = pl.pallas_call(
        body,
        grid=(1,),
        in_specs=in_specs,
        out_specs=[
            pl.BlockSpec((B * N, 1), lambda i: (0, 0)),
            pl.BlockSpec((B * N, 1), lambda i: (0, 0)),
        ],
        out_shape=[
            jax.ShapeDtypeStruct((B * N, 1), jnp.float32),
            jax.ShapeDtypeStruct((B * N, 1), jnp.float32),
        ],
    )(*ins)

    return jnp.concatenate([out0, out1], axis=1).reshape(B, N, 2)


# speedup vs baseline: 6.0545x; 5.6377x over previous
import functools

import jax
import jax.numpy as jnp
from jax import lax
from jax.experimental import pallas as pl
from jax.experimental.pallas import tpu as pltpu
from jax.experimental.pallas import tpu_sc as plsc


def _sc_gather(table4, src_pad, dst_pad, *, B, N, RE, Ep):
    mesh = plsc.VectorSubcoreMesh(core_axis_name="c", subcore_axis_name="s")
    NC = mesh.num_cores

    @functools.partial(
        pl.kernel,
        out_type=jax.ShapeDtypeStruct((B, Ep, RE), jnp.float32),
        mesh=mesh,
        scratch_types=[
            pltpu.VMEM((Ep,), jnp.int32),
            pltpu.VMEM((Ep,), jnp.int32),
            pltpu.VMEM((N, RE, N), jnp.float32),
            pltpu.VMEM((Ep, RE), jnp.float32),
        ],
        compiler_params=pltpu.CompilerParams(
            needs_layout_passes=False, use_tc_tiling_on_sc=False),
    )
    def k(table_h, src_h, dst_h, out_h, src_v, dst_v, stage_v, out_v):
        wid = lax.axis_index("s") * NC + lax.axis_index("c")

        @pl.when(wid < B)
        def _():
            pltpu.sync_copy(src_h, src_v)
            pltpu.sync_copy(dst_h, dst_v)
            pltpu.sync_copy(table_h.at[pl.ds(wid * N, N)], stage_v)
            lanes = lax.iota(jnp.int32, 16)
            for c in range(Ep // 16):
                s16 = src_v[pl.ds(c * 16, 16)]
                d16 = dst_v[pl.ds(c * 16, 16)]
                e16 = lanes + (c * 16)
                for r in range(RE):
                    rvec = jnp.full((16,), r, jnp.int32)
                    vals = plsc.load_gather(stage_v, [s16, rvec, d16])
                    plsc.store_scatter(out_v, [e16, rvec], vals)
            pltpu.sync_copy(out_v, out_h.at[wid])

    return k(table4, src_pad, dst_pad)


def _tc_body(refs, B, N, Ep):
    (dn_r, dl_r, ge_r, graph_r, srcc_r, dstc_r, t2p_r,
     Wne, bne, Wle, ble, Wgn, Wgl, Wee, bee, We2n,
     Wn, Wl, We, Wdn, Wzn, Wde, Wze, Wdl, Wzl,
     Wdno, bdno, Wdeo, bdeo, Wdlo, bdlo,
     W1n, W1l, W1e, bs1, wdiff, bdiff,
     out0_r, out1_r) = refs

    def dot(a, b):
        return jnp.dot(a, b, preferred_element_type=jnp.float32)

    def dot_c00(a, b):
        return lax.dot_general(a, b, (((0,), (0,)), ((), ())),
                               preferred_element_type=jnp.float32)

    BN = B * N
    dn = dn_r[...].reshape(BN, dn_r.shape[-1])
    dl = dl_r[...].reshape(BN, dl_r.shape[-1])
    ge = ge_r[...]

    xn = dot(dn, Wne[...]) + bne[...]
    xl = dot(dl, Wle[...]) + ble[...]
    xe = dot(ge, Wee[...]) + bee[...]

    s_dn = dn[0:N]
    s_dl = dl[0:N]
    for b in range(1, B):
        s_dn = s_dn + dn[b * N:(b + 1) * N]
        s_dl = s_dl + dl[b * N:(b + 1) * N]
    s_node = dot(s_dn / B, Wne[...]) + bne[...]
    s_log = dot(s_dl / B, Wle[...]) + ble[...]
    h = jnp.tanh(dot(s_node, Wgn[...]) + dot(s_log, Wgl[...]))
    hhT = lax.dot_general(h, h, (((1,), (1,)), ((), ())),
                          preferred_element_type=jnp.float32)
    ewmat = (1.0 / (1.0 + jnp.exp(-hhT))) * graph_r[...]

    iota_e = lax.broadcasted_iota(jnp.int32, (Ep, N), 1)
    oh_src = (srcc_r[...] == iota_e).astype(jnp.float32)
    oh_dst = (dstc_r[...] == iota_e).astype(jnp.float32)
    ew_col = jnp.sum(dot(oh_src, ewmat) * oh_dst, axis=1, keepdims=True)

    t2p = t2p_r[...]

    ew_full = jnp.concatenate([ew_col] * B, axis=0)
    e_fl = xe * ew_full
    msg = dot(e_fl, We2n[...])
    z_edge = jnp.tanh(dot(e_fl, We[...]))
    edge = jnp.tanh(dot(xe, Wde[...]) + dot(z_edge, Wze[...]))
    rec1 = jnp.square(dot(edge, Wdeo[...]) + bdeo[...] - ge)

    aggs = []
    rec_edges = []
    for b in range(B):
        aggs.append(dot_c00(oh_dst, msg[b * Ep:(b + 1) * Ep]))
        rec_edges.append(dot_c00(t2p, rec1[b * Ep:(b + 1) * Ep]))
    agg = jnp.concatenate(aggs, axis=0)
    rec_edge = jnp.concatenate(rec_edges, axis=0)

    z_node = jnp.tanh(dot(xn, Wn[...]) + agg)
    z_log = jnp.tanh(dot(xl, Wl[...]))
    node = jnp.tanh(dot(xn, Wdn[...]) + dot(z_node, Wzn[...]))
    logf = jnp.tanh(dot(xl, Wdl[...]) + dot(z_log, Wzl[...]))
    rec_node = jnp.square(dot(node, Wdno[...]) + bdno[...] - dn)
    rec_log = jnp.square(dot(logf, Wdlo[...]) + bdlo[...] - dl)

    pre = (dot(rec_node, W1n[...]) + dot(rec_log, W1l[...])
           + dot(rec_edge, W1e[...]) + bs1[...])
    hshow = jnp.where(pre > 0, pre, 0.01 * pre)
    delta = dot(hshow, wdiff[...]) + bdiff[...]
    out0_r[...] = 1.0 / (1.0 + jnp.exp(-delta))
    out1_r[...] = 1.0 / (1.0 + jnp.exp(delta))


def kernel(data_node, data_edge, data_log, groundtruth_cls, graph, src, dst,
           trace2pod, params):
    p = params
    B, W, N, RN = data_node.shape
    RE = data_edge.shape[-1]
    LL = data_log.shape[-1]
    E = src.shape[0]
    Ep = ((E + 127) // 128) * 128

    src_pad = jnp.zeros((Ep,), jnp.int32).at[:E].set(src.astype(jnp.int32))
    dst_pad = jnp.zeros((Ep,), jnp.int32).at[:E].set(dst.astype(jnp.int32))

    table4 = jnp.swapaxes(data_edge[:, -1], 2, 3).reshape(B * N, RE, N)
    ge = _sc_gather(table4, src_pad, dst_pad, B=B, N=N, RE=RE, Ep=Ep)
    ge2 = ge.reshape(B * Ep, RE)

    src_col = jnp.full((Ep, 1), N, jnp.int32).at[:E, 0].set(src.astype(jnp.int32))
    dst_col = jnp.full((Ep, 1), N, jnp.int32).at[:E, 0].set(dst.astype(jnp.int32))
    t2p_pad = jnp.zeros((Ep, N), jnp.float32).at[:E].set(trace2pod)

    r2 = lambda v: v.reshape(1, -1)
    ins = [
        data_node, data_log, ge2, graph, src_col, dst_col, t2p_pad,
        p['Wne'], r2(p['bne']), p['Wle'], r2(p['ble']),
        p['Wgn'], p['Wgl'], p['Wee'], r2(p['bee']), p['We2n'],
        p['Wn'], p['Wl'], p['We'], p['Wdn'], p['Wzn'],
        p['Wde'], p['Wze'], p['Wdl'], p['Wzl'],
        p['Wdn_out'], r2(p['bdn_out']), p['Wde_out'], r2(p['bde_out']),
        p['Wdl_out'], r2(p['bdl_out']),
        p['Ws1'][:RN], p['Ws1'][RN:RN + LL], p['Ws1'][RN + LL:], r2(p['bs1']),
        (p['Ws2'][:, 0] - p['Ws2'][:, 1]).reshape(-1, 1),
        (p['bs2'][0] - p['bs2'][1]).reshape(1, 1),
    ]

    def zero_map(nd):
        return lambda i: (0,) * nd

    in_specs = [
        pl.BlockSpec((B, 1, N, RN), lambda i: (0, W - 1, 0, 0)),
        pl.BlockSpec((B, 1, N, LL), lambda i: (0, W - 1, 0, 0)),
    ] + [pl.BlockSpec(x.shape, zero_map(x.ndim)) for x in ins[2:]]

    body = lambda *refs: _tc_body(refs, B, N, Ep)
    out0, out1 = pl.pallas_call(
        body,
        grid=(1,),
        in_specs=in_specs,
        out_specs=[
            pl.BlockSpec((B * N, 1), lambda i: (0, 0)),
            pl.BlockSpec((B * N, 1), lambda i: (0, 0)),
        ],
        out_shape=[
            jax.ShapeDtypeStruct((B * N, 1), jnp.float32),
            jax.ShapeDtypeStruct((B * N, 1), jnp.float32),
        ],
    )(*ins)

    return jnp.concatenate([out0, out1], axis=1).reshape(B, N, 2)
